# baseline (speedup 1.0000x reference)
import jax
import jax.numpy as jnp
from jax.experimental import pallas as pl


def _gcn_conv(x, src, dst, ew, W, b):
    n = x.shape[0]
    loop = jnp.arange(n, dtype=src.dtype)
    s = jnp.concatenate([src, loop])
    d = jnp.concatenate([dst, loop])
    w = jnp.concatenate([ew, jnp.ones((n,), x.dtype)])
    deg = jnp.zeros((n,), x.dtype).at[d].add(w)
    dinv = jnp.where(deg > 0, jax.lax.rsqrt(deg), 0.0)
    norm = dinv[s] * w * dinv[d]
    h = x @ W
    out = jnp.zeros_like(h).at[d].add(h[s] * norm[:, None])
    return out + b


def _final_linear_kernel(z_ref, wf_ref, bf_ref, o_ref):
    o_ref[...] = z_ref[...] @ wf_ref[...] + bf_ref[...]


def kernel(x, edge_index, edge_attr, batch, metadata, W1, b1, W2, b2, Wm, bm, Wf, bf):
    src, dst = edge_index[0], edge_index[1]
    h = jax.nn.relu(_gcn_conv(x, src, dst, edge_attr, W1, b1))
    h = jax.nn.relu(_gcn_conv(h, src, dst, edge_attr, W2, b2))
    G = metadata.shape[0]
    sums = jax.ops.segment_sum(h, batch, num_segments=G)
    counts = jax.ops.segment_sum(jnp.ones((h.shape[0],), h.dtype), batch, num_segments=G)
    pooled = sums / jnp.maximum(counts, 1.0)[:, None]
    first_node = jnp.searchsorted(batch, jnp.arange(G, dtype=batch.dtype))
    idx = first_node % metadata.shape[0]
    md = metadata[idx]
    md = jax.nn.relu(md @ Wm + bm)
    z = jnp.concatenate([pooled, md], axis=1)
    out = pl.pallas_call(
        _final_linear_kernel,
        out_shape=jax.ShapeDtypeStruct((G, Wf.shape[1]), jnp.float32),
    )(z, Wf, bf)
    return out


# speedup vs baseline: 8.2070x; 8.2070x over previous
import functools

import jax
import jax.numpy as jnp
from jax import lax
from jax.experimental import pallas as pl
from jax.experimental.pallas import tpu as pltpu
from jax.experimental.pallas import tpu_sc as plsc

_NC = 2
_NS = 16
_W = 128


def _deg_body(dst_hbm, ew_hbm, out_hbm, idx_v, ew_v, stage_v, acc_sh):
    c = lax.axis_index("c")
    s = lax.axis_index("s")
    n = stage_v.shape[0]
    rows = idx_v.shape[0]

    @pl.when(s == 0)
    def _zero():
        def zb(i, carry):
            stage_v[pl.ds(i * 16, 16)] = jnp.zeros((16,), jnp.float32)
            return carry
        lax.fori_loop(0, n // 16, zb, 0)
        pltpu.sync_copy(stage_v, acc_sh)

    plsc.subcore_barrier()

    base = (c * _NS + s) * rows
    pltpu.sync_copy(dst_hbm.at[pl.ds(base, rows)], idx_v)
    pltpu.sync_copy(ew_hbm.at[pl.ds(base, rows)], ew_v)

    def body(j, carry):
        pltpu.sync_copy(ew_v.at[j], acc_sh.at[idx_v.at[j]], add=True)
        return carry
    lax.fori_loop(0, rows, body, 0)

    plsc.subcore_barrier()

    @pl.when(s == 0)
    def _writeback():
        pltpu.sync_copy(acc_sh, stage_v)
        pltpu.sync_copy(stage_v, out_hbm.at[pl.ds(c * n, n)])


def _deg_call(dst2d, ew2d, n2):
    rows = dst2d.shape[0] // (_NC * _NS)
    mesh = plsc.VectorSubcoreMesh(core_axis_name="c", subcore_axis_name="s")
    kfn = pl.kernel(
        functools.partial(_deg_body),
        mesh=mesh,
        out_type=jax.ShapeDtypeStruct((_NC * n2,), jnp.float32),
        scratch_types=[
            pltpu.VMEM((rows, _W), jnp.int32),
            pltpu.VMEM((rows, _W), jnp.float32),
            pltpu.VMEM((n2,), jnp.float32),
            pltpu.VMEM_SHARED((n2,), jnp.float32),
        ],
    )
    return kfn(dst2d, ew2d)


def _msg_body(u_hbm, src_hbm, dst_hbm, ew_hbm, out_hbm,
              srcv, dstv, ewv, rows_v, sem, acc_sh):
    c = lax.axis_index("c")
    s = lax.axis_index("s")
    rows = srcv.shape[0]
    stripe = acc_sh.shape[0] // _NS
    sizes = [_W] * (stripe // _W) + ([stripe % _W] if stripe % _W else [])

    def zb(i, carry):
        for q in range(8):
            rows_v[i, pl.ds(q * 16, 16)] = jnp.zeros((16,), jnp.float32)
        return carry
    lax.fori_loop(0, _W, zb, 0)
    off = 0
    for sz in sizes:
        pltpu.sync_copy(rows_v.at[pl.ds(0, sz)],
                        acc_sh.at[pl.ds(s * stripe + off, sz)])
        off += sz

    plsc.subcore_barrier()

    base = (c * _NS + s) * rows
    pltpu.sync_copy(src_hbm.at[pl.ds(base, rows)], srcv)
    pltpu.sync_copy(dst_hbm.at[pl.ds(base, rows)], dstv)
    pltpu.sync_copy(ew_hbm.at[pl.ds(base, rows)], ewv)

    def chunk(j, carry):
        pltpu.async_copy(u_hbm.at[srcv.at[j]], rows_v, sem).wait()

        def grp(eb, c2):
            wv = ewv[j, pl.ds(eb * 16, 16)]
            for l in range(16):
                wb = lax.broadcast_in_dim(wv[l], (16,), ())
                e = eb * 16 + l
                for q in range(8):
                    rows_v[e, pl.ds(q * 16, 16)] = (
                        rows_v[e, pl.ds(q * 16, 16)] * wb)
            return c2
        lax.fori_loop(0, _W // 16, grp, 0)

        pltpu.sync_copy(rows_v, acc_sh.at[dstv.at[j]], add=True)
        return carry
    lax.fori_loop(0, rows, chunk, 0)

    plsc.subcore_barrier()

    off = 0
    for sz in sizes:
        pltpu.sync_copy(acc_sh.at[pl.ds(s * stripe + off, sz)],
                        rows_v.at[pl.ds(0, sz)])
        pltpu.sync_copy(rows_v.at[pl.ds(0, sz)],
                        out_hbm.at[c, pl.ds(s * stripe + off, sz)])
        off += sz


def _msg_call(u, src2d, dst2d, ew2d, n2):
    n, d = u.shape
    rows = src2d.shape[0] // (_NC * _NS)
    stripe = n2 // _NS
    mesh = plsc.VectorSubcoreMesh(core_axis_name="c", subcore_axis_name="s")
    kfn = pl.kernel(
        functools.partial(_msg_body),
        mesh=mesh,
        out_type=jax.ShapeDtypeStruct((_NC, n2, d), jnp.float32),
        scratch_types=[
            pltpu.VMEM((rows, _W), jnp.int32),
            pltpu.VMEM((rows, _W), jnp.int32),
            pltpu.VMEM((rows, _W), jnp.float32),
            pltpu.VMEM((_W, d), jnp.float32),
            pltpu.SemaphoreType.DMA,
            pltpu.VMEM_SHARED((n2, d), jnp.float32),
        ],
    )
    return kfn(u, src2d, dst2d, ew2d)


def _k2_body(degp_ref, x_ref, w_ref, u_ref, dinv_ref):
    deg = degp_ref[0] + degp_ref[1] + 1.0
    dinv = jnp.where(deg > 0, lax.rsqrt(deg), 0.0)
    dinv_ref[...] = dinv
    u_ref[...] = jnp.dot(x_ref[...], w_ref[...],
                         preferred_element_type=jnp.float32) * dinv[:, None]


def _k4_body(mp_ref, u_ref, dinv_ref, b_ref, w_ref, u2_ref):
    dinv = dinv_ref[...]
    h = (mp_ref[0] + mp_ref[1] + u_ref[...]) * dinv[:, None] + b_ref[...][None, :]
    h = jnp.maximum(h, 0.0)
    u2_ref[...] = jnp.dot(h, w_ref[...],
                          preferred_element_type=jnp.float32) * dinv[:, None]


def _k5_body(mp_ref, u_ref, dinv_ref, b_ref, batch_ref, md_ref,
             wm_ref, bm_ref, wfp_ref, wfm_ref, bf_ref, o_ref):
    n = u_ref.shape[0]
    g = md_ref.shape[0]
    dinv = dinv_ref[...]
    h = (mp_ref[0] + mp_ref[1] + u_ref[...]) * dinv[:, None] + b_ref[...][None, :]
    h = jnp.maximum(h, 0.0)
    bt = batch_ref[...]
    gi = lax.broadcasted_iota(jnp.int32, (g, n), 0)
    onehot = (gi == bt[None, :]).astype(jnp.float32)
    sums = jnp.dot(onehot, h, preferred_element_type=jnp.float32)
    counts = jnp.sum(onehot, axis=1)
    pooled = sums / jnp.maximum(counts, 1.0)[:, None]
    first_node = jnp.sum((bt[None, :] < gi).astype(jnp.int32), axis=1)
    idx = lax.rem(first_node, g)
    oh2 = (lax.broadcasted_iota(jnp.int32, (g, g), 1) == idx[:, None]
           ).astype(jnp.float32)
    md = jnp.dot(oh2, md_ref[...], preferred_element_type=jnp.float32)
    md = jnp.maximum(jnp.dot(md, wm_ref[...],
                             preferred_element_type=jnp.float32)
                     + bm_ref[...][None, :], 0.0)
    o_ref[...] = (jnp.dot(pooled, wfp_ref[...], preferred_element_type=jnp.float32)
                  + jnp.dot(md, wfm_ref[...], preferred_element_type=jnp.float32)
                  + bf_ref[...][None, :])


def kernel(x, edge_index, edge_attr, batch, metadata, W1, b1, W2, b2,
           Wm, bm, Wf, bf):
    n, d = x.shape
    e = edge_attr.shape[0]
    g = metadata.shape[0]
    chunk = _NC * _NS * _W
    rpt = -(-e // chunk)
    rpt = ((rpt + 7) // 8) * 8
    ep = rpt * chunk
    pad = ep - e
    src2d = jnp.concatenate(
        [edge_index[0], jnp.zeros((pad,), edge_index.dtype)]).reshape(-1, _W)
    dst2d = jnp.concatenate(
        [edge_index[1], jnp.zeros((pad,), edge_index.dtype)]).reshape(-1, _W)
    ew2d = jnp.concatenate(
        [edge_attr, jnp.zeros((pad,), edge_attr.dtype)]).reshape(-1, _W)

    n2 = ((n + _W - 1) // _W) * _W
    degp = _deg_call(dst2d, ew2d, n2).reshape(_NC, n2)[:, :n]

    u1, dinv = pl.pallas_call(
        _k2_body,
        out_shape=[jax.ShapeDtypeStruct((n, d), jnp.float32),
                   jax.ShapeDtypeStruct((n,), jnp.float32)],
    )(degp, x, W1)

    m1 = _msg_call(u1, src2d, dst2d, ew2d, n2)[:, :n]

    u2 = pl.pallas_call(
        _k4_body,
        out_shape=jax.ShapeDtypeStruct((n, d), jnp.float32),
    )(m1, u1, dinv, b1, W2)

    m2 = _msg_call(u2, src2d, dst2d, ew2d, n2)[:, :n]

    out = pl.pallas_call(
        _k5_body,
        out_shape=jax.ShapeDtypeStruct((g, Wf.shape[1]), jnp.float32),
    )(m2, u2, dinv, b2, batch, metadata, Wm, bm, Wf[:d], Wf[d:], bf)
    return out


# speedup vs baseline: 9.6117x; 1.1712x over previous
import functools

import jax
import jax.numpy as jnp
from jax import lax
from jax.experimental import pallas as pl
from jax.experimental.pallas import tpu as pltpu
from jax.experimental.pallas import tpu_sc as plsc

_NC = 2
_NS = 16
_W = 128


def _deg_body(dst_hbm, ew_hbm, out_hbm, idx_v, ew_v, stage_v, acc_sh):
    c = lax.axis_index("c")
    s = lax.axis_index("s")
    n = stage_v.shape[0]
    rows = idx_v.shape[0]

    @pl.when(s == 0)
    def _zero():
        def zb(i, carry):
            stage_v[pl.ds(i * 16, 16)] = jnp.zeros((16,), jnp.float32)
            return carry
        lax.fori_loop(0, n // 16, zb, 0)
        pltpu.sync_copy(stage_v, acc_sh)

    plsc.subcore_barrier()

    base = (c * _NS + s) * rows
    pltpu.sync_copy(dst_hbm.at[pl.ds(base, rows)], idx_v)
    pltpu.sync_copy(ew_hbm.at[pl.ds(base, rows)], ew_v)

    def body(j, carry):
        pltpu.sync_copy(ew_v.at[j], acc_sh.at[idx_v.at[j]], add=True)
        return carry
    lax.fori_loop(0, rows, body, 0)

    plsc.subcore_barrier()

    @pl.when(s == 0)
    def _writeback():
        pltpu.sync_copy(acc_sh, stage_v)
        pltpu.sync_copy(stage_v, out_hbm.at[pl.ds(c * n, n)])


def _deg_call(dst2d, ew2d, n2):
    rows = dst2d.shape[0] // (_NC * _NS)
    mesh = plsc.VectorSubcoreMesh(core_axis_name="c", subcore_axis_name="s")
    kfn = pl.kernel(
        functools.partial(_deg_body),
        mesh=mesh,
        out_type=jax.ShapeDtypeStruct((_NC * n2,), jnp.float32),
        scratch_types=[
            pltpu.VMEM((rows, _W), jnp.int32),
            pltpu.VMEM((rows, _W), jnp.float32),
            pltpu.VMEM((n2,), jnp.float32),
            pltpu.VMEM_SHARED((n2,), jnp.float32),
        ],
    )
    return kfn(dst2d, ew2d)


def _scale(buf, ewv, r):
    def grp(eb, c2):
        wv = ewv[r, pl.ds(eb * 16, 16)]
        for l in range(16):
            wb = lax.broadcast_in_dim(wv[l], (16,), ())
            e = eb * 16 + l
            for q in range(8):
                buf[e, pl.ds(q * 16, 16)] = buf[e, pl.ds(q * 16, 16)] * wb
        return c2
    lax.fori_loop(0, _W // 16, grp, 0)


def _msg_body(u_hbm, src_hbm, dst_hbm, ew_hbm, out_hbm,
              srcv, dstv, ewv, rows_a, rows_b, gs_a, gs_b, ss_a, ss_b,
              acc_sh):
    c = lax.axis_index("c")
    s = lax.axis_index("s")
    blk = srcv.shape[0]
    rows = src_hbm.shape[0] // (_NC * _NS)
    nblk = rows // blk
    pairs = blk // 2
    stripe = acc_sh.shape[0] // _NS
    sizes = [_W] * (stripe // _W) + ([stripe % _W] if stripe % _W else [])

    def zb(i, carry):
        for q in range(8):
            rows_a[i, pl.ds(q * 16, 16)] = jnp.zeros((16,), jnp.float32)
        return carry
    lax.fori_loop(0, _W, zb, 0)
    off = 0
    for sz in sizes:
        pltpu.sync_copy(rows_a.at[pl.ds(0, sz)],
                        acc_sh.at[pl.ds(s * stripe + off, sz)])
        off += sz
    plsc.subcore_barrier()

    base = (c * _NS + s) * rows
    dummy = u_hbm.at[pl.ds(0, _W)]

    def wait_d(sem, buf):
        pltpu.make_async_copy(dummy, buf, sem).wait()

    def block(b, carry):
        @pl.when(b > 0)
        def _drain():
            wait_d(ss_a, rows_a)
            wait_d(ss_b, rows_b)
        pltpu.sync_copy(src_hbm.at[pl.ds(base + b * blk, blk)], srcv)
        pltpu.sync_copy(dst_hbm.at[pl.ds(base + b * blk, blk)], dstv)
        pltpu.sync_copy(ew_hbm.at[pl.ds(base + b * blk, blk)], ewv)
        pltpu.async_copy(u_hbm.at[srcv.at[0]], rows_a, gs_a)

        def pair(p, c2):
            r0 = 2 * p
            r1 = 2 * p + 1
            wait_d(gs_a, rows_a)

            @pl.when(p > 0)
            def _wb():
                wait_d(ss_b, rows_b)
            pltpu.async_copy(u_hbm.at[srcv.at[r1]], rows_b, gs_b)
            _scale(rows_a, ewv, r0)
            pltpu.async_copy(rows_a, acc_sh.at[dstv.at[r0]], ss_a, add=True)
            wait_d(gs_b, rows_b)

            @pl.when(p < pairs - 1)
            def _wa():
                wait_d(ss_a, rows_a)
                pltpu.async_copy(u_hbm.at[srcv.at[r0 + 2]], rows_a, gs_a)
            _scale(rows_b, ewv, r1)
            pltpu.async_copy(rows_b, acc_sh.at[dstv.at[r1]], ss_b, add=True)
            return c2
        lax.fori_loop(0, pairs, pair, 0)
        return carry
    lax.fori_loop(0, nblk, block, 0)

    wait_d(ss_a, rows_a)
    wait_d(ss_b, rows_b)
    plsc.subcore_barrier()

    off = 0
    for sz in sizes:
        pltpu.sync_copy(acc_sh.at[pl.ds(s * stripe + off, sz)],
                        rows_a.at[pl.ds(0, sz)])
        pltpu.sync_copy(rows_a.at[pl.ds(0, sz)],
                        out_hbm.at[c, pl.ds(s * stripe + off, sz)])
        off += sz


def _msg_call(u, src2d, dst2d, ew2d, n2):
    n, d = u.shape
    rows = src2d.shape[0] // (_NC * _NS)
    blk = rows // 2
    mesh = plsc.VectorSubcoreMesh(core_axis_name="c", subcore_axis_name="s")
    kfn = pl.kernel(
        functools.partial(_msg_body),
        mesh=mesh,
        out_type=jax.ShapeDtypeStruct((_NC, n2, d), jnp.float32),
        scratch_types=[
            pltpu.VMEM((blk, _W), jnp.int32),
            pltpu.VMEM((blk, _W), jnp.int32),
            pltpu.VMEM((blk, _W), jnp.float32),
            pltpu.VMEM((_W, d), jnp.float32),
            pltpu.VMEM((_W, d), jnp.float32),
            pltpu.SemaphoreType.DMA,
            pltpu.SemaphoreType.DMA,
            pltpu.SemaphoreType.DMA,
            pltpu.SemaphoreType.DMA,
            pltpu.VMEM_SHARED((n2, d), jnp.float32),
        ],
    )
    return kfn(u, src2d, dst2d, ew2d)


def _k2_body(degp_ref, x_ref, w_ref, u_ref, dinv_ref):
    deg = degp_ref[0] + degp_ref[1] + 1.0
    dinv = jnp.where(deg > 0, lax.rsqrt(deg), 0.0)
    dinv_ref[...] = dinv
    u_ref[...] = jnp.dot(x_ref[...], w_ref[...],
                         preferred_element_type=jnp.float32) * dinv[:, None]


def _k4_body(mp_ref, u_ref, dinv_ref, b_ref, w_ref, u2_ref):
    dinv = dinv_ref[...]
    h = (mp_ref[0] + mp_ref[1] + u_ref[...]) * dinv[:, None] + b_ref[...][None, :]
    h = jnp.maximum(h, 0.0)
    u2_ref[...] = jnp.dot(h, w_ref[...],
                          preferred_element_type=jnp.float32) * dinv[:, None]


def _k5_body(mp_ref, u_ref, dinv_ref, b_ref, batch_ref, md_ref,
             wm_ref, bm_ref, wfp_ref, wfm_ref, bf_ref, o_ref):
    n = u_ref.shape[0]
    g = md_ref.shape[0]
    dinv = dinv_ref[...]
    h = (mp_ref[0] + mp_ref[1] + u_ref[...]) * dinv[:, None] + b_ref[...][None, :]
    h = jnp.maximum(h, 0.0)
    bt = batch_ref[...]
    gi = lax.broadcasted_iota(jnp.int32, (g, n), 0)
    onehot = (gi == bt[None, :]).astype(jnp.float32)
    sums = jnp.dot(onehot, h, preferred_element_type=jnp.float32)
    counts = jnp.sum(onehot, axis=1)
    pooled = sums / jnp.maximum(counts, 1.0)[:, None]
    first_node = jnp.sum((bt[None, :] < gi).astype(jnp.int32), axis=1)
    idx = lax.rem(first_node, g)
    oh2 = (lax.broadcasted_iota(jnp.int32, (g, g), 1) == idx[:, None]
           ).astype(jnp.float32)
    md = jnp.dot(oh2, md_ref[...], preferred_element_type=jnp.float32)
    md = jnp.maximum(jnp.dot(md, wm_ref[...],
                             preferred_element_type=jnp.float32)
                     + bm_ref[...][None, :], 0.0)
    o_ref[...] = (jnp.dot(pooled, wfp_ref[...], preferred_element_type=jnp.float32)
                  + jnp.dot(md, wfm_ref[...], preferred_element_type=jnp.float32)
                  + bf_ref[...][None, :])


def kernel(x, edge_index, edge_attr, batch, metadata, W1, b1, W2, b2,
           Wm, bm, Wf, bf):
    n, d = x.shape
    e = edge_attr.shape[0]
    g = metadata.shape[0]
    chunk = _NC * _NS * _W
    rpt = -(-e // chunk)
    rpt = ((rpt + 7) // 8) * 8
    ep = rpt * chunk
    pad = ep - e
    src2d = jnp.concatenate(
        [edge_index[0], jnp.zeros((pad,), edge_index.dtype)]).reshape(-1, _W)
    dst2d = jnp.concatenate(
        [edge_index[1], jnp.zeros((pad,), edge_index.dtype)]).reshape(-1, _W)
    ew2d = jnp.concatenate(
        [edge_attr, jnp.zeros((pad,), edge_attr.dtype)]).reshape(-1, _W)

    n2 = ((n + _W - 1) // _W) * _W
    degp = _deg_call(dst2d, ew2d, n2).reshape(_NC, n2)[:, :n]

    u1, dinv = pl.pallas_call(
        _k2_body,
        out_shape=[jax.ShapeDtypeStruct((n, d), jnp.float32),
                   jax.ShapeDtypeStruct((n,), jnp.float32)],
    )(degp, x, W1)

    m1 = _msg_call(u1, src2d, dst2d, ew2d, n2)[:, :n]

    u2 = pl.pallas_call(
        _k4_body,
        out_shape=jax.ShapeDtypeStruct((n, d), jnp.float32),
    )(m1, u1, dinv, b1, W2)

    m2 = _msg_call(u2, src2d, dst2d, ew2d, n2)[:, :n]

    out = pl.pallas_call(
        _k5_body,
        out_shape=jax.ShapeDtypeStruct((g, Wf.shape[1]), jnp.float32),
    )(m2, u2, dinv, b2, batch, metadata, Wm, bm, Wf[:d], Wf[d:], bf)
    return out


# speedup vs baseline: 10.4257x; 1.0847x over previous
import functools

import jax
import jax.numpy as jnp
from jax import lax
from jax.experimental import pallas as pl
from jax.experimental.pallas import tpu as pltpu
from jax.experimental.pallas import tpu_sc as plsc

_NC = 2
_NS = 16
_W = 128
_K0 = 3
_K1 = 1


def _deg_body(dst_hbm, ew_hbm, out_hbm, idx_v, ew_v, stage_v, acc_sh):
    c = lax.axis_index("c")
    s = lax.axis_index("s")
    n = stage_v.shape[0]
    rows = idx_v.shape[0]

    @pl.when(s == 0)
    def _zero():
        def zb(i, carry):
            stage_v[pl.ds(i * 16, 16)] = jnp.zeros((16,), jnp.float32)
            return carry
        lax.fori_loop(0, n // 16, zb, 0)
        pltpu.sync_copy(stage_v, acc_sh)

    plsc.subcore_barrier()

    base = (c * _NS + s) * rows
    pltpu.sync_copy(dst_hbm.at[pl.ds(base, rows)], idx_v)
    pltpu.sync_copy(ew_hbm.at[pl.ds(base, rows)], ew_v)

    def body(j, carry):
        pltpu.sync_copy(ew_v.at[j], acc_sh.at[idx_v.at[j]], add=True)
        return carry
    lax.fori_loop(0, rows, body, 0)

    plsc.subcore_barrier()

    @pl.when(s == 0)
    def _writeback():
        pltpu.sync_copy(acc_sh, stage_v)
        pltpu.sync_copy(stage_v, out_hbm.at[pl.ds(c * n, n)])


def _deg_call(dst2d, ew2d, n2):
    rows = dst2d.shape[0] // (_NC * _NS)
    mesh = plsc.VectorSubcoreMesh(core_axis_name="c", subcore_axis_name="s")
    kfn = pl.kernel(
        functools.partial(_deg_body),
        mesh=mesh,
        out_type=jax.ShapeDtypeStruct((_NC * n2,), jnp.float32),
        scratch_types=[
            pltpu.VMEM((rows, _W), jnp.int32),
            pltpu.VMEM((rows, _W), jnp.float32),
            pltpu.VMEM((n2,), jnp.float32),
            pltpu.VMEM_SHARED((n2,), jnp.float32),
        ],
    )
    return kfn(dst2d, ew2d)


def _scale(buf, ewv, r):
    def grp(eb, c2):
        wv = ewv[r, pl.ds(eb * 16, 16)]
        for l in range(16):
            wb = lax.broadcast_in_dim(wv[l], (16,), ())
            e = eb * 16 + l
            for q in range(8):
                buf[e, pl.ds(q * 16, 16)] = buf[e, pl.ds(q * 16, 16)] * wb
        return c2
    lax.fori_loop(0, _W // 16, grp, 0)


def _msg_body(u_hbm, src_hbm, dst_hbm, ew_hbm, out_hbm,
              srcv, dstv, ewv, rows_a, rows_b, gs_a, gs_b, ss_a, ss_b,
              acc_sh):
    c = lax.axis_index("c")
    s = lax.axis_index("s")
    blk = srcv.shape[0]
    pairs = blk // 2
    nblk = jnp.where(c == 0, _K0, _K1)
    stripe = acc_sh.shape[0] // _NS
    sizes = [_W] * (stripe // _W) + ([stripe % _W] if stripe % _W else [])

    def zb(i, carry):
        for q in range(8):
            rows_a[i, pl.ds(q * 16, 16)] = jnp.zeros((16,), jnp.float32)
        return carry
    lax.fori_loop(0, _W, zb, 0)
    off = 0
    for sz in sizes:
        pltpu.sync_copy(rows_a.at[pl.ds(0, sz)],
                        acc_sh.at[pl.ds(s * stripe + off, sz)])
        off += sz
    plsc.subcore_barrier()

    base = jnp.where(c == 0, _K0 * s, _K0 * _NS + s) * blk
    dummy = u_hbm.at[pl.ds(0, _W)]

    def wait_d(sem, buf):
        pltpu.make_async_copy(dummy, buf, sem).wait()

    def block(b, carry):
        @pl.when(b > 0)
        def _drain():
            wait_d(ss_a, rows_a)
            wait_d(ss_b, rows_b)
        pltpu.sync_copy(src_hbm.at[pl.ds(base + b * blk, blk)], srcv)
        pltpu.sync_copy(dst_hbm.at[pl.ds(base + b * blk, blk)], dstv)
        pltpu.sync_copy(ew_hbm.at[pl.ds(base + b * blk, blk)], ewv)
        pltpu.async_copy(u_hbm.at[srcv.at[0]], rows_a, gs_a)

        def pair(p, c2):
            r0 = 2 * p
            r1 = 2 * p + 1
            wait_d(gs_a, rows_a)

            @pl.when(p > 0)
            def _wb():
                wait_d(ss_b, rows_b)
            pltpu.async_copy(u_hbm.at[srcv.at[r1]], rows_b, gs_b)
            _scale(rows_a, ewv, r0)
            pltpu.async_copy(rows_a, acc_sh.at[dstv.at[r0]], ss_a, add=True)
            wait_d(gs_b, rows_b)

            @pl.when(p < pairs - 1)
            def _wa():
                wait_d(ss_a, rows_a)
                pltpu.async_copy(u_hbm.at[srcv.at[r0 + 2]], rows_a, gs_a)
            _scale(rows_b, ewv, r1)
            pltpu.async_copy(rows_b, acc_sh.at[dstv.at[r1]], ss_b, add=True)
            return c2
        lax.fori_loop(0, pairs, pair, 0)
        return carry
    lax.fori_loop(0, nblk, block, 0)

    wait_d(ss_a, rows_a)
    wait_d(ss_b, rows_b)
    plsc.subcore_barrier()

    off = 0
    for sz in sizes:
        pltpu.sync_copy(acc_sh.at[pl.ds(s * stripe + off, sz)],
                        rows_a.at[pl.ds(0, sz)])
        pltpu.sync_copy(rows_a.at[pl.ds(0, sz)],
                        out_hbm.at[c, pl.ds(s * stripe + off, sz)])
        off += sz


def _msg_call(u, src2d, dst2d, ew2d, n2):
    n, d = u.shape
    blk = src2d.shape[0] // (_NS * (_K0 + _K1))
    mesh = plsc.VectorSubcoreMesh(core_axis_name="c", subcore_axis_name="s")
    kfn = pl.kernel(
        functools.partial(_msg_body),
        mesh=mesh,
        out_type=jax.ShapeDtypeStruct((_NC, n2, d), jnp.float32),
        scratch_types=[
            pltpu.VMEM((blk, _W), jnp.int32),
            pltpu.VMEM((blk, _W), jnp.int32),
            pltpu.VMEM((blk, _W), jnp.float32),
            pltpu.VMEM((_W, d), jnp.float32),
            pltpu.VMEM((_W, d), jnp.float32),
            pltpu.SemaphoreType.DMA,
            pltpu.SemaphoreType.DMA,
            pltpu.SemaphoreType.DMA,
            pltpu.SemaphoreType.DMA,
            pltpu.VMEM_SHARED((n2, d), jnp.float32),
        ],
    )
    return kfn(u, src2d, dst2d, ew2d)


def _k2_body(degp_ref, x_ref, w_ref, u_ref, dinv_ref):
    deg = degp_ref[0] + degp_ref[1] + 1.0
    dinv = jnp.where(deg > 0, lax.rsqrt(deg), 0.0)
    dinv_ref[...] = dinv
    u_ref[...] = jnp.dot(x_ref[...], w_ref[...],
                         preferred_element_type=jnp.float32) * dinv[:, None]


def _k4_body(mp_ref, u_ref, dinv_ref, b_ref, w_ref, u2_ref):
    dinv = dinv_ref[...]
    h = (mp_ref[0] + mp_ref[1] + u_ref[...]) * dinv[:, None] + b_ref[...][None, :]
    h = jnp.maximum(h, 0.0)
    u2_ref[...] = jnp.dot(h, w_ref[...],
                          preferred_element_type=jnp.float32) * dinv[:, None]


def _k5_body(mp_ref, u_ref, dinv_ref, b_ref, batch_ref, md_ref,
             wm_ref, bm_ref, wfp_ref, wfm_ref, bf_ref, o_ref):
    n = u_ref.shape[0]
    g = md_ref.shape[0]
    dinv = dinv_ref[...]
    h = (mp_ref[0] + mp_ref[1] + u_ref[...]) * dinv[:, None] + b_ref[...][None, :]
    h = jnp.maximum(h, 0.0)
    bt = batch_ref[...]
    gi = lax.broadcasted_iota(jnp.int32, (g, n), 0)
    onehot = (gi == bt[None, :]).astype(jnp.float32)
    sums = jnp.dot(onehot, h, preferred_element_type=jnp.float32)
    counts = jnp.sum(onehot, axis=1)
    pooled = sums / jnp.maximum(counts, 1.0)[:, None]
    first_node = jnp.sum((bt[None, :] < gi).astype(jnp.int32), axis=1)
    idx = lax.rem(first_node, g)
    oh2 = (lax.broadcasted_iota(jnp.int32, (g, g), 1) == idx[:, None]
           ).astype(jnp.float32)
    md = jnp.dot(oh2, md_ref[...], preferred_element_type=jnp.float32)
    md = jnp.maximum(jnp.dot(md, wm_ref[...],
                             preferred_element_type=jnp.float32)
                     + bm_ref[...][None, :], 0.0)
    o_ref[...] = (jnp.dot(pooled, wfp_ref[...], preferred_element_type=jnp.float32)
                  + jnp.dot(md, wfm_ref[...], preferred_element_type=jnp.float32)
                  + bf_ref[...][None, :])


def kernel(x, edge_index, edge_attr, batch, metadata, W1, b1, W2, b2,
           Wm, bm, Wf, bf):
    n, d = x.shape
    e = edge_attr.shape[0]
    g = metadata.shape[0]
    chunk = _NC * _NS * _W
    rpt = -(-e // chunk)
    rpt = ((rpt + 15) // 16) * 16
    ep = rpt * chunk
    pad = ep - e
    src2d = jnp.concatenate(
        [edge_index[0], jnp.zeros((pad,), edge_index.dtype)]).reshape(-1, _W)
    dst2d = jnp.concatenate(
        [edge_index[1], jnp.zeros((pad,), edge_index.dtype)]).reshape(-1, _W)
    ew2d = jnp.concatenate(
        [edge_attr, jnp.zeros((pad,), edge_attr.dtype)]).reshape(-1, _W)

    n2 = ((n + _W - 1) // _W) * _W
    degp = _deg_call(dst2d, ew2d, n2).reshape(_NC, n2)[:, :n]

    u1, dinv = pl.pallas_call(
        _k2_body,
        out_shape=[jax.ShapeDtypeStruct((n, d), jnp.float32),
                   jax.ShapeDtypeStruct((n,), jnp.float32)],
    )(degp, x, W1)

    m1 = _msg_call(u1, src2d, dst2d, ew2d, n2)[:, :n]

    u2 = pl.pallas_call(
        _k4_body,
        out_shape=jax.ShapeDtypeStruct((n, d), jnp.float32),
    )(m1, u1, dinv, b1, W2)

    m2 = _msg_call(u2, src2d, dst2d, ew2d, n2)[:, :n]

    out = pl.pallas_call(
        _k5_body,
        out_shape=jax.ShapeDtypeStruct((g, Wf.shape[1]), jnp.float32),
    )(m2, u2, dinv, b2, batch, metadata, Wm, bm, Wf[:d], Wf[d:], bf)
    return out
